# baseline (speedup 1.0000x reference)
import functools

import jax
import jax.numpy as jnp
from jax import lax
from jax.experimental import pallas as pl
from jax.experimental.pallas import tpu as pltpu
from jax.experimental.pallas import tpu_sc as plsc

_CP = pltpu.CompilerParams(use_tc_tiling_on_sc=True, needs_layout_passes=False)


def _make_repack(vocab: int, emb: int):
    info = plsc.get_sparse_core_info()
    nc, ns = info.num_cores, info.num_subcores
    nw = nc * ns
    CH = 2 * emb
    n_reg = (vocab // CH) * CH // CH
    tail = vocab - n_reg * CH
    slots = ((n_reg + nw - 1) // nw + 1) // 2 * 2
    half = slots // 2

    mesh = plsc.VectorSubcoreMesh(core_axis_name="c", subcore_axis_name="s")

    @functools.partial(
        pl.kernel,
        mesh=mesh,
        out_type=jax.ShapeDtypeStruct((vocab // 2, 2 * emb), jnp.float32),
        scratch_types=[
            pltpu.VMEM((emb, CH), jnp.float32),
            pltpu.VMEM((emb, CH), jnp.float32),
            pltpu.VMEM((emb, CH), jnp.float32),
            pltpu.VMEM((emb, CH), jnp.float32),
            pltpu.VMEM((emb, emb), jnp.float32),
            pltpu.VMEM((emb // 2, CH), jnp.float32),
            pltpu.SemaphoreType.DMA,
            pltpu.SemaphoreType.DMA,
            pltpu.SemaphoreType.DMA,
            pltpu.SemaphoreType.DMA,
        ],
        compiler_params=_CP,
    )
    def repack_kernel(wt_hbm, w2_hbm, blk0, blk1, tblk0, tblk1,
                      tb, ttb, lsem0, lsem1, ssem0, ssem1):
        wid = lax.axis_index("s") * nc + lax.axis_index("c")
        iota = lax.iota(jnp.int32, 16)

        def t_of(i):
            return i * nw + wid

        def valid(i):
            return t_of(i) < n_reg

        def load(i, blk, sem, start):
            cp = pltpu.make_async_copy(
                wt_hbm.at[:, pl.ds(t_of(i) * CH, CH)], blk, sem)
            cp.start() if start else cp.wait()

        def transpose(blk, tblk):
            for u in range(emb):
                for k in range(8):
                    rowk = iota + 16 * (k % 4)
                    colk = jnp.full((16,), 2 * u + (1 if k >= 4 else 0),
                                    jnp.int32)
                    tblk[u, pl.ds(16 * k, 16)] = plsc.load_gather(
                        blk, [rowk, colk])

        def store(i, tblk, sem, start):
            cp = pltpu.make_async_copy(
                tblk, w2_hbm.at[pl.ds(t_of(i) * emb, emb), :], sem)
            cp.start() if start else cp.wait()

        @pl.when(valid(0))
        def _():
            load(0, blk0, lsem0, True)

        @pl.when(valid(1))
        def _():
            load(1, blk1, lsem1, True)

        def body(j, carry):
            a = 2 * j

            @pl.when(valid(a))
            def _():
                load(a, blk0, lsem0, False)

            @pl.when((j >= 1) & valid(a - 2))
            def _():
                store(a - 2, tblk0, ssem0, False)

            @pl.when(valid(a))
            def _():
                transpose(blk0, tblk0)
                store(a, tblk0, ssem0, True)

            @pl.when(valid(a + 2))
            def _():
                load(a + 2, blk0, lsem0, True)

            @pl.when(valid(a + 1))
            def _():
                load(a + 1, blk1, lsem1, False)

            @pl.when((j >= 1) & valid(a - 1))
            def _():
                store(a - 1, tblk1, ssem1, False)

            @pl.when(valid(a + 1))
            def _():
                transpose(blk1, tblk1)
                store(a + 1, tblk1, ssem1, True)

            @pl.when(valid(a + 3))
            def _():
                load(a + 3, blk1, lsem1, True)

            return carry

        lax.fori_loop(0, half, body, 0)

        @pl.when(valid(slots - 2))
        def _():
            store(slots - 2, tblk0, ssem0, False)

        @pl.when(valid(slots - 1))
        def _():
            store(slots - 1, tblk1, ssem1, False)

        if tail:
            @pl.when(wid == 0)
            def _():
                pltpu.sync_copy(wt_hbm.at[:, pl.ds(n_reg * CH, tail)], tb)
                for u in range(tail // 2):
                    for k in range(8):
                        rowk = iota + 16 * (k % 4)
                        colk = jnp.full((16,), 2 * u + (1 if k >= 4 else 0),
                                        jnp.int32)
                        ttb[u, pl.ds(16 * k, 16)] = plsc.load_gather(
                            tb, [rowk, colk])
                pltpu.sync_copy(
                    ttb, w2_hbm.at[pl.ds(n_reg * emb, tail // 2), :])

    return repack_kernel


def _make_lookup(vocab: int, emb: int, b_dim: int, l_dim: int):
    info = plsc.get_sparse_core_info()
    nc, ns = info.num_cores, info.num_subcores
    nw = nc * ns
    CH = 128
    n_bc = b_dim // CH
    n_chunks = l_dim * n_bc
    per_w = n_chunks // nw
    assert n_chunks % (nw * 2) == 0
    half = per_w // 2

    mesh = plsc.VectorSubcoreMesh(core_axis_name="c", subcore_axis_name="s")

    @functools.partial(
        pl.kernel,
        mesh=mesh,
        out_type=jax.ShapeDtypeStruct((l_dim, emb, b_dim), jnp.float32),
        scratch_types=[
            pltpu.VMEM((CH,), jnp.int32),
            pltpu.VMEM((CH,), jnp.int32),
            pltpu.VMEM((CH,), jnp.int32),
            pltpu.VMEM((CH,), jnp.int32),
            pltpu.VMEM((CH,), jnp.int32),
            pltpu.VMEM((CH,), jnp.int32),
            pltpu.VMEM((CH, 2 * emb), jnp.float32),
            pltpu.VMEM((CH, 2 * emb), jnp.float32),
            pltpu.VMEM((emb, CH), jnp.float32),
            pltpu.VMEM((emb, CH), jnp.float32),
            pltpu.SemaphoreType.DMA,
            pltpu.SemaphoreType.DMA,
            pltpu.SemaphoreType.DMA,
            pltpu.SemaphoreType.DMA,
            pltpu.SemaphoreType.DMA,
            pltpu.SemaphoreType.DMA,
        ],
        compiler_params=_CP,
    )
    def lookup_kernel(mt_hbm, w2_hbm, out_hbm,
                      idxraw0, idxraw1, gidx0, gidx1, par0, par1,
                      rows0, rows1, trows0, trows1,
                      isem0, isem1, gsem0, gsem1, osem0, osem1):
        wid = lax.axis_index("s") * nc + lax.axis_index("c")
        base = wid * per_w
        iota = lax.iota(jnp.int32, 16)

        def lc(t):
            return lax.div(t, n_bc), lax.rem(t, n_bc) * CH

        def idx_load(t, dst, sem, start):
            l, c = lc(t)
            cp = pltpu.make_async_copy(
                mt_hbm.at[pl.ds(l * b_dim + c, CH)], dst, sem)
            cp.start() if start else cp.wait()

        def process(idxraw, gidx, par):
            for k in range(0, CH, 16):
                v = idxraw[pl.ds(k, 16)]
                gidx[pl.ds(k, 16)] = lax.shift_right_logical(v, 1)
                par[pl.ds(k, 16)] = lax.shift_left(jnp.bitwise_and(v, 1), 6)

        def gather(gidx, rows, sem, start):
            cp = pltpu.make_async_copy(w2_hbm.at[gidx], rows, sem)
            cp.start() if start else cp.wait()

        def transpose(rows, par, trows):
            for k in range(0, CH, 16):
                rowk = iota + k
                pk = par[pl.ds(k, 16)]
                for e in range(emb):
                    trows[e, pl.ds(k, 16)] = plsc.load_gather(
                        rows, [rowk, pk + e])

        def store(t, trows, sem, start):
            l, c = lc(t)
            cp = pltpu.make_async_copy(
                trows, out_hbm.at[l, :, pl.ds(c, CH)], sem)
            cp.start() if start else cp.wait()

        idx_load(base, idxraw0, isem0, True)
        idx_load(base, idxraw0, isem0, False)
        process(idxraw0, gidx0, par0)
        gather(gidx0, rows0, gsem0, True)
        idx_load(base + 1, idxraw1, isem1, True)

        def body(j, carry):
            a = base + 2 * j

            idx_load(a + 1, idxraw1, isem1, False)
            process(idxraw1, gidx1, par1)
            gather(gidx1, rows1, gsem1, True)

            @pl.when(2 * j + 2 < per_w)
            def _():
                idx_load(a + 2, idxraw0, isem0, True)

            gather(gidx0, rows0, gsem0, False)

            @pl.when(j >= 1)
            def _():
                store(a - 2, trows0, osem0, False)

            transpose(rows0, par0, trows0)
            store(a, trows0, osem0, True)

            @pl.when(2 * j + 2 < per_w)
            def _():
                idx_load(a + 2, idxraw0, isem0, False)
                process(idxraw0, gidx0, par0)
                gather(gidx0, rows0, gsem0, True)

            @pl.when(2 * j + 3 < per_w)
            def _():
                idx_load(a + 3, idxraw1, isem1, True)

            gather(gidx1, rows1, gsem1, False)

            @pl.when(j >= 1)
            def _():
                store(a - 1, trows1, osem1, False)

            transpose(rows1, par1, trows1)
            store(a + 1, trows1, osem1, True)
            return carry

        lax.fori_loop(0, half, body, 0)

        store(base + per_w - 2, trows0, osem0, False)
        store(base + per_w - 1, trows1, osem1, False)

    return lookup_kernel


def kernel(mask, weight):
    b, l = mask.shape
    vocab, emb = weight.shape
    mtf = mask.T.reshape(-1)
    wt = weight.T
    w2 = _make_repack(vocab, emb)(wt)
    out = _make_lookup(vocab, emb, b, l)(mtf, w2)
    return out.transpose(2, 0, 1)


# speedup vs baseline: 1.4431x; 1.4431x over previous
import functools

import jax
import jax.numpy as jnp
from jax import lax
from jax.experimental import pallas as pl
from jax.experimental.pallas import tpu as pltpu
from jax.experimental.pallas import tpu_sc as plsc

_CP = pltpu.CompilerParams(use_tc_tiling_on_sc=True, needs_layout_passes=False)


def _make_repack(vocab: int, emb: int):
    info = plsc.get_sparse_core_info()
    nc, ns = info.num_cores, info.num_subcores
    nw = nc * ns
    CH = 2 * emb
    n_reg = (vocab // CH) * CH // CH
    tail = vocab - n_reg * CH
    slots = ((n_reg + nw - 1) // nw + 1) // 2 * 2
    half = slots // 2

    mesh = plsc.VectorSubcoreMesh(core_axis_name="c", subcore_axis_name="s")

    @functools.partial(
        pl.kernel,
        mesh=mesh,
        out_type=jax.ShapeDtypeStruct((vocab // 2, 2 * emb), jnp.float32),
        scratch_types=[
            pltpu.VMEM((emb, CH), jnp.float32),
            pltpu.VMEM((emb, CH), jnp.float32),
            pltpu.VMEM((emb, CH), jnp.float32),
            pltpu.VMEM((emb, CH), jnp.float32),
            pltpu.VMEM((emb, emb), jnp.float32),
            pltpu.VMEM((emb // 2, CH), jnp.float32),
            pltpu.SemaphoreType.DMA,
            pltpu.SemaphoreType.DMA,
            pltpu.SemaphoreType.DMA,
            pltpu.SemaphoreType.DMA,
        ],
        compiler_params=_CP,
    )
    def repack_kernel(wt_hbm, w2_hbm, blk0, blk1, tblk0, tblk1,
                      tb, ttb, lsem0, lsem1, ssem0, ssem1):
        wid = lax.axis_index("s") * nc + lax.axis_index("c")
        iota = lax.iota(jnp.int32, 16)

        def t_of(i):
            return i * nw + wid

        def valid(i):
            return t_of(i) < n_reg

        def load(i, blk, sem, start):
            cp = pltpu.make_async_copy(
                wt_hbm.at[:, pl.ds(t_of(i) * CH, CH)], blk, sem)
            cp.start() if start else cp.wait()

        def transpose(blk, tblk):
            for u in range(emb):
                tvs = [
                    plsc.load_gather(
                        blk,
                        [iota + 16 * (k % 4),
                         jnp.full((16,), 2 * u + (1 if k >= 4 else 0),
                                  jnp.int32)])
                    for k in range(8)
                ]
                for k in range(8):
                    tblk[u, pl.ds(16 * k, 16)] = tvs[k]

        def store(i, tblk, sem, start):
            cp = pltpu.make_async_copy(
                tblk, w2_hbm.at[pl.ds(t_of(i) * emb, emb), :], sem)
            cp.start() if start else cp.wait()

        @pl.when(valid(0))
        def _():
            load(0, blk0, lsem0, True)

        @pl.when(valid(1))
        def _():
            load(1, blk1, lsem1, True)

        def body(j, carry):
            a = 2 * j

            @pl.when(valid(a))
            def _():
                load(a, blk0, lsem0, False)

            @pl.when((j >= 1) & valid(a - 2))
            def _():
                store(a - 2, tblk0, ssem0, False)

            @pl.when(valid(a))
            def _():
                transpose(blk0, tblk0)
                store(a, tblk0, ssem0, True)

            @pl.when(valid(a + 2))
            def _():
                load(a + 2, blk0, lsem0, True)

            @pl.when(valid(a + 1))
            def _():
                load(a + 1, blk1, lsem1, False)

            @pl.when((j >= 1) & valid(a - 1))
            def _():
                store(a - 1, tblk1, ssem1, False)

            @pl.when(valid(a + 1))
            def _():
                transpose(blk1, tblk1)
                store(a + 1, tblk1, ssem1, True)

            @pl.when(valid(a + 3))
            def _():
                load(a + 3, blk1, lsem1, True)

            return carry

        lax.fori_loop(0, half, body, 0)

        @pl.when(valid(slots - 2))
        def _():
            store(slots - 2, tblk0, ssem0, False)

        @pl.when(valid(slots - 1))
        def _():
            store(slots - 1, tblk1, ssem1, False)

        if tail:
            @pl.when(wid == 0)
            def _():
                pltpu.sync_copy(wt_hbm.at[:, pl.ds(n_reg * CH, tail)], tb)
                for u in range(tail // 2):
                    for k in range(8):
                        rowk = iota + 16 * (k % 4)
                        colk = jnp.full((16,), 2 * u + (1 if k >= 4 else 0),
                                        jnp.int32)
                        ttb[u, pl.ds(16 * k, 16)] = plsc.load_gather(
                            tb, [rowk, colk])
                pltpu.sync_copy(
                    ttb, w2_hbm.at[pl.ds(n_reg * emb, tail // 2), :])

    return repack_kernel


def _make_lookup(vocab: int, emb: int, b_dim: int, l_dim: int):
    info = plsc.get_sparse_core_info()
    nc, ns = info.num_cores, info.num_subcores
    nw = nc * ns
    CH = 128
    n_bc = b_dim // CH
    n_chunks = l_dim * n_bc
    per_w = n_chunks // nw
    assert n_chunks % (nw * 2) == 0
    half = per_w // 2

    mesh = plsc.VectorSubcoreMesh(core_axis_name="c", subcore_axis_name="s")

    @functools.partial(
        pl.kernel,
        mesh=mesh,
        out_type=jax.ShapeDtypeStruct((l_dim, emb, b_dim), jnp.float32),
        scratch_types=[
            pltpu.VMEM((CH,), jnp.int32),
            pltpu.VMEM((CH,), jnp.int32),
            pltpu.VMEM((CH,), jnp.int32),
            pltpu.VMEM((CH,), jnp.int32),
            pltpu.VMEM((CH,), jnp.int32),
            pltpu.VMEM((CH,), jnp.int32),
            pltpu.VMEM((CH, 2 * emb), jnp.float32),
            pltpu.VMEM((CH, 2 * emb), jnp.float32),
            pltpu.VMEM((emb, CH), jnp.float32),
            pltpu.VMEM((emb, CH), jnp.float32),
            pltpu.SemaphoreType.DMA,
            pltpu.SemaphoreType.DMA,
            pltpu.SemaphoreType.DMA,
            pltpu.SemaphoreType.DMA,
            pltpu.SemaphoreType.DMA,
            pltpu.SemaphoreType.DMA,
        ],
        compiler_params=_CP,
    )
    def lookup_kernel(mt_hbm, w2_hbm, out_hbm,
                      idxraw0, idxraw1, gidx0, gidx1, par0, par1,
                      rows0, rows1, trows0, trows1,
                      isem0, isem1, gsem0, gsem1, osem0, osem1):
        wid = lax.axis_index("s") * nc + lax.axis_index("c")
        base = wid * per_w
        iota = lax.iota(jnp.int32, 16)

        def lc(t):
            return lax.div(t, n_bc), lax.rem(t, n_bc) * CH

        def idx_load(t, dst, sem, start):
            l, c = lc(t)
            cp = pltpu.make_async_copy(
                mt_hbm.at[pl.ds(l * b_dim + c, CH)], dst, sem)
            cp.start() if start else cp.wait()

        def process(idxraw, gidx, par):
            for k in range(0, CH, 16):
                v = idxraw[pl.ds(k, 16)]
                gidx[pl.ds(k, 16)] = lax.shift_right_logical(v, 1)
                par[pl.ds(k, 16)] = lax.shift_left(jnp.bitwise_and(v, 1), 6)

        def gather(gidx, rows, sem, start):
            cp = pltpu.make_async_copy(w2_hbm.at[gidx], rows, sem)
            cp.start() if start else cp.wait()

        def transpose(rows, par, trows):
            for k in range(0, CH, 16):
                rowk = iota + k
                pk = par[pl.ds(k, 16)]
                for e0 in range(0, emb, 8):
                    tvs = [
                        plsc.load_gather(rows, [rowk, pk + (e0 + d)])
                        for d in range(8)
                    ]
                    for d in range(8):
                        trows[e0 + d, pl.ds(k, 16)] = tvs[d]

        def store(t, trows, sem, start):
            l, c = lc(t)
            cp = pltpu.make_async_copy(
                trows, out_hbm.at[l, :, pl.ds(c, CH)], sem)
            cp.start() if start else cp.wait()

        idx_load(base, idxraw0, isem0, True)
        idx_load(base, idxraw0, isem0, False)
        process(idxraw0, gidx0, par0)
        gather(gidx0, rows0, gsem0, True)
        idx_load(base + 1, idxraw1, isem1, True)

        def body(j, carry):
            a = base + 2 * j

            idx_load(a + 1, idxraw1, isem1, False)
            process(idxraw1, gidx1, par1)
            gather(gidx1, rows1, gsem1, True)

            @pl.when(2 * j + 2 < per_w)
            def _():
                idx_load(a + 2, idxraw0, isem0, True)

            gather(gidx0, rows0, gsem0, False)

            @pl.when(j >= 1)
            def _():
                store(a - 2, trows0, osem0, False)

            transpose(rows0, par0, trows0)
            store(a, trows0, osem0, True)

            @pl.when(2 * j + 2 < per_w)
            def _():
                idx_load(a + 2, idxraw0, isem0, False)
                process(idxraw0, gidx0, par0)
                gather(gidx0, rows0, gsem0, True)

            @pl.when(2 * j + 3 < per_w)
            def _():
                idx_load(a + 3, idxraw1, isem1, True)

            gather(gidx1, rows1, gsem1, False)

            @pl.when(j >= 1)
            def _():
                store(a - 1, trows1, osem1, False)

            transpose(rows1, par1, trows1)
            store(a + 1, trows1, osem1, True)
            return carry

        lax.fori_loop(0, half, body, 0)

        store(base + per_w - 2, trows0, osem0, False)
        store(base + per_w - 1, trows1, osem1, False)

    return lookup_kernel


def kernel(mask, weight):
    b, l = mask.shape
    vocab, emb = weight.shape
    mtf = mask.T.reshape(-1)
    wt = weight.T
    w2 = _make_repack(vocab, emb)(wt)
    out = _make_lookup(vocab, emb, b, l)(mtf, w2)
    return out.transpose(2, 0, 1)
